# baseline (speedup 1.0000x reference)
import functools

import jax
import jax.numpy as jnp
from jax import lax
from jax.experimental import pallas as pl
from jax.experimental.pallas import tpu as pltpu
from jax.experimental.pallas import tpu_sc as plsc

VOCAB = 1000000
N_EMBD = 64
SEQ_LEN = 2048
BATCH = 4

TILE_W = 128
LAST_FULL = (VOCAB // TILE_W) * TILE_W - TILE_W
SAFE_MAX = LAST_FULL + TILE_W - 1
TAIL_BASE = LAST_FULL + TILE_W
LANES = 16
SEGS = N_EMBD // LANES

T_SC = 1024
T_TC = SEQ_LEN - T_SC

NC = 2
NS = 16
NW = NC * NS
B_SC = BATCH * T_SC
BPW = B_SC // NW
WPB = NW // BATCH
QUAD = 4
NQUAD = 8
RING = 2 * QUAD
VGRP = BPW // LANES
PGRP = BPW // (QUAD * NQUAD)

_mesh = plsc.VectorSubcoreMesh(core_axis_name="c", subcore_axis_name="s")


@functools.partial(
    pl.kernel,
    out_type=jax.ShapeDtypeStruct((BATCH, N_EMBD, T_SC), jnp.float32),
    mesh=_mesh,
    scratch_types=[
        pltpu.VMEM((BPW,), jnp.int32),
        pltpu.VMEM((RING, N_EMBD, TILE_W), jnp.float32),
        pltpu.VMEM((N_EMBD, BPW), jnp.float32),
        pltpu.VMEM((N_EMBD, BPW), jnp.float32),
        pltpu.VMEM((N_EMBD, VOCAB - TAIL_BASE), jnp.float32),
        pltpu.SemaphoreType.DMA,
        pltpu.SemaphoreType.DMA,
    ],
    compiler_params=pltpu.CompilerParams(
        use_tc_tiling_on_sc=True, needs_layout_passes=False),
)
def _embed_sc(xf_hbm, tokT_hbm, posT_hbm, out_hbm, idx_v, blocks_v, fbuf,
              pbuf, tail_v, sem_a, sem_b):
    wid = lax.axis_index("s") * NC + lax.axis_index("c")
    base = pl.multiple_of(wid * BPW, BPW)
    b = wid // WPB
    t0 = pl.multiple_of(lax.rem(wid, WPB) * BPW, BPW)

    pltpu.sync_copy(xf_hbm.at[pl.ds(base, BPW)], idx_v)
    pltpu.sync_copy(posT_hbm.at[:, pl.ds(t0, BPW)], pbuf)

    lane_iota = lax.iota(jnp.int32, LANES)

    def extract(src_ref, ring_slot, lane, k):
        lane_s = jnp.full((LANES,), lane, jnp.int32)
        k_s = jnp.full((LANES,), k, jnp.int32)
        for f in range(SEGS):
            feat = lane_iota + (f * LANES)
            if ring_slot is None:
                tv = plsc.load_gather(src_ref, [feat, lane_s])
            else:
                slot_s = jnp.full((LANES,), ring_slot, jnp.int32)
                tv = plsc.load_gather(src_ref, [slot_s, feat, lane_s])
            pv = plsc.load_gather(pbuf, [feat, k_s])
            plsc.store_scatter(fbuf, [feat, k_s], tv + pv)

    sems = (sem_a, sem_b)

    def group(g, _):
        vec_a = jnp.minimum(idx_v[pl.ds(g * 2 * LANES, LANES)], SAFE_MAX)
        vec_b = jnp.minimum(
            idx_v[pl.ds(g * 2 * LANES + LANES, LANES)], SAFE_MAX)

        def quad_tok(q, j):
            lane = q * QUAD + j
            vec = vec_a if lane < LANES else vec_b
            return vec[lane % LANES]

        def fire(q):
            half = (q % 2) * QUAD
            cps = []
            for j in range(QUAD):
                tok = quad_tok(q, j)
                blk = pl.multiple_of((tok >> 7) * TILE_W, TILE_W)
                cps.append(pltpu.async_copy(
                    tokT_hbm.at[:, pl.ds(blk, TILE_W)],
                    blocks_v.at[half + j], sems[q % 2]))
            return cps

        pending = fire(0)
        for q in range(NQUAD):
            nxt = fire(q + 1) if q + 1 < NQUAD else None
            for c in pending:
                c.wait()
            half = (q % 2) * QUAD
            for j in range(QUAD):
                tok = quad_tok(q, j)
                extract(blocks_v, half + j, tok & (TILE_W - 1),
                        g * QUAD * NQUAD + q * QUAD + j)
            pending = nxt
        return 0

    lax.fori_loop(0, PGRP, group, 0)

    def tail_group(g, _):
        idx_vec = idx_v[pl.ds(g * LANES, LANES)]
        any_tail = jnp.max(idx_vec) >= TAIL_BASE

        @pl.when(any_tail)
        def _():
            pltpu.sync_copy(tokT_hbm.at[:, pl.ds(TAIL_BASE, VOCAB - TAIL_BASE)],
                            tail_v)
            for j in range(LANES):
                tok = idx_vec[j]

                @pl.when(tok >= TAIL_BASE)
                def _():
                    extract(tail_v, None, tok - TAIL_BASE, g * LANES + j)

        return 0

    lax.fori_loop(0, VGRP, tail_group, 0)

    pltpu.sync_copy(fbuf, out_hbm.at[b, :, pl.ds(t0, BPW)])


TCB = 64
B_TC = BATCH * T_TC
TC_STEPS = B_TC // TCB
STEPS_PER_B = T_TC // TCB


STRIP_W = TCB * TILE_W + (VOCAB - TAIL_BASE)


def _tc_body(xs_ref, idx_ref, posT_ref, tokT_ref, tail_ref, out_ref, big, sem):
    s = pl.program_id(0)
    n = pl.num_programs(0)

    def fire(step, slot):
        for j in range(TCB):
            blk = pl.multiple_of(xs_ref[step * TCB + j], TILE_W)
            pltpu.make_async_copy(
                tokT_ref.at[:, pl.ds(blk, TILE_W)],
                big.at[slot, :, pl.ds(j * TILE_W, TILE_W)],
                sem.at[slot],
            ).start()

    def wait(slot):
        pltpu.make_async_copy(
            tokT_ref.at[:, pl.ds(0, TCB * TILE_W)],
            big.at[slot],
            sem.at[slot],
        ).wait()

    @pl.when(s == 0)
    def _():
        fire(0, 0)

    @pl.when(s + 1 < n)
    def _():
        fire(s + 1, (s + 1) % 2)

    wait(s % 2)

    ids = idx_ref[0, 0, :]
    safe = jnp.minimum(ids, SAFE_MAX)
    col = jnp.where(
        ids >= TAIL_BASE,
        TCB * TILE_W + ids - TAIL_BASE,
        (safe & (TILE_W - 1)) + lax.iota(jnp.int32, TCB) * TILE_W)
    onehot = (lax.broadcasted_iota(jnp.int32, (TCB, STRIP_W), 1)
              == col[:, None]).astype(jnp.float32)
    strip = jnp.concatenate([big[s % 2], tail_ref[...]], axis=1)
    cols = lax.dot_general(strip, onehot, (((1,), (1,)), ((), ())),
                           preferred_element_type=jnp.float32)
    out_ref[0, :, :] = cols + posT_ref[0]


@functools.partial(
    pl.pallas_call,
    grid_spec=pltpu.PrefetchScalarGridSpec(
        num_scalar_prefetch=1,
        grid=(TC_STEPS,),
        in_specs=[
            pl.BlockSpec((1, 1, TCB), lambda s, xs: (s, 0, 0)),
            pl.BlockSpec((1, N_EMBD, TCB), lambda s, xs: (s % STEPS_PER_B, 0, 0)),
            pl.BlockSpec(memory_space=pl.ANY),
            pl.BlockSpec((N_EMBD, VOCAB - TAIL_BASE), lambda s, xs: (0, 0)),
        ],
        out_specs=pl.BlockSpec((1, N_EMBD, TCB), lambda s, xs: (s, 0, 0)),
        scratch_shapes=[
            pltpu.VMEM((2, N_EMBD, TCB * TILE_W), jnp.float32),
            pltpu.SemaphoreType.DMA((2,)),
        ],
    ),
    out_shape=jax.ShapeDtypeStruct((TC_STEPS, N_EMBD, TCB), jnp.float32),
)
def _embed_tc(xs_ref, idx_ref, posT_ref, tokT_ref, tail_ref, out_ref, big, sem):
    _tc_body(xs_ref, idx_ref, posT_ref, tokT_ref, tail_ref, out_ref, big, sem)


def kernel(x, tok_table, pos_table):
    x = x.astype(jnp.int32)
    tokT = tok_table.T
    posT = pos_table.T
    x_sc = x[:, :T_SC].reshape(-1)
    x_tc = x[:, T_SC:].reshape(-1)
    pos3 = posT[:, T_SC:].reshape(N_EMBD, STEPS_PER_B, TCB).transpose(1, 0, 2)
    out_sc = _embed_sc(x_sc, tokT, posT[:, :T_SC])
    blk_tc = (jnp.minimum(x_tc, SAFE_MAX) >> 7) * TILE_W
    out_tc = _embed_tc(blk_tc, x_tc.reshape(TC_STEPS, 1, TCB), pos3, tokT,
                       tokT[:, TAIL_BASE:])
    out_tc = (out_tc.reshape(BATCH, STEPS_PER_B, N_EMBD, TCB)
              .transpose(0, 2, 1, 3).reshape(BATCH, N_EMBD, T_TC))
    out_fm = jnp.concatenate([out_sc, out_tc], axis=2)
    return out_fm.transpose(0, 2, 1)


# speedup vs baseline: 1.0320x; 1.0320x over previous
import functools

import jax
import jax.numpy as jnp
from jax import lax
from jax.experimental import pallas as pl
from jax.experimental.pallas import tpu as pltpu
from jax.experimental.pallas import tpu_sc as plsc

VOCAB = 1000000
N_EMBD = 64
SEQ_LEN = 2048
BATCH = 4

TILE_W = 128
LAST_FULL = (VOCAB // TILE_W) * TILE_W - TILE_W
SAFE_MAX = LAST_FULL + TILE_W - 1
TAIL_BASE = LAST_FULL + TILE_W
LANES = 16
SEGS = N_EMBD // LANES

T_SC = 1280
T_TC = SEQ_LEN - T_SC
PWIN = 384

NC = 2
NS = 16
NW = NC * NS
B_SC = BATCH * T_SC
BPW = B_SC // NW
WPB = NW // BATCH
QUAD = 4
NQUAD = 8
RING = 2 * QUAD
VGRP = BPW // LANES
PGRP = BPW // (QUAD * NQUAD)

_mesh = plsc.VectorSubcoreMesh(core_axis_name="c", subcore_axis_name="s")


@functools.partial(
    pl.kernel,
    out_type=jax.ShapeDtypeStruct((NW, N_EMBD, BPW), jnp.float32),
    mesh=_mesh,
    scratch_types=[
        pltpu.VMEM((BPW,), jnp.int32),
        pltpu.VMEM((RING, N_EMBD, TILE_W), jnp.float32),
        pltpu.VMEM((N_EMBD, BPW), jnp.float32),
        pltpu.VMEM((N_EMBD, PWIN), jnp.float32),
        pltpu.VMEM((N_EMBD, VOCAB - TAIL_BASE), jnp.float32),
        pltpu.SemaphoreType.DMA,
        pltpu.SemaphoreType.DMA,
    ],
    compiler_params=pltpu.CompilerParams(
        use_tc_tiling_on_sc=True, needs_layout_passes=False),
)
def _embed_sc(xf_hbm, tokT_hbm, posT_hbm, out_hbm, idx_v, blocks_v, fbuf,
              pbuf, tail_v, sem_a, sem_b):
    wid = lax.axis_index("s") * NC + lax.axis_index("c")
    base = pl.multiple_of(wid * BPW, 8)
    t0 = lax.rem(wid, WPB) * BPW
    wbase = pl.multiple_of((t0 // TILE_W) * TILE_W, TILE_W)
    woff = t0 - wbase

    pltpu.sync_copy(xf_hbm.at[pl.ds(base, BPW)], idx_v)
    pltpu.sync_copy(posT_hbm.at[:, pl.ds(wbase, PWIN)], pbuf)

    lane_iota = lax.iota(jnp.int32, LANES)

    def extract(src_ref, ring_slot, lane, k):
        lane_s = jnp.full((LANES,), lane, jnp.int32)
        k_s = jnp.full((LANES,), k, jnp.int32)
        for f in range(SEGS):
            feat = lane_iota + (f * LANES)
            if ring_slot is None:
                tv = plsc.load_gather(src_ref, [feat, lane_s])
            else:
                slot_s = jnp.full((LANES,), ring_slot, jnp.int32)
                tv = plsc.load_gather(src_ref, [slot_s, feat, lane_s])
            pv = plsc.load_gather(pbuf, [feat, k_s + woff])
            plsc.store_scatter(fbuf, [feat, k_s], tv + pv)

    sems = (sem_a, sem_b)

    def group(g, _):
        vec_a = jnp.minimum(idx_v[pl.ds(g * 2 * LANES, LANES)], SAFE_MAX)
        vec_b = jnp.minimum(
            idx_v[pl.ds(g * 2 * LANES + LANES, LANES)], SAFE_MAX)

        def quad_tok(q, j):
            lane = q * QUAD + j
            vec = vec_a if lane < LANES else vec_b
            return vec[lane % LANES]

        def fire(q):
            half = (q % 2) * QUAD
            cps = []
            for j in range(QUAD):
                tok = quad_tok(q, j)
                blk = pl.multiple_of((tok >> 7) * TILE_W, TILE_W)
                cps.append(pltpu.async_copy(
                    tokT_hbm.at[:, pl.ds(blk, TILE_W)],
                    blocks_v.at[half + j], sems[q % 2]))
            return cps

        pending = fire(0)
        for q in range(NQUAD):
            nxt = fire(q + 1) if q + 1 < NQUAD else None
            for c in pending:
                c.wait()
            half = (q % 2) * QUAD
            for j in range(QUAD):
                tok = quad_tok(q, j)
                extract(blocks_v, half + j, tok & (TILE_W - 1),
                        g * QUAD * NQUAD + q * QUAD + j)
            pending = nxt
        return 0

    lax.fori_loop(0, PGRP, group, 0)

    def tail_group(g, _):
        idx_vec = idx_v[pl.ds(g * LANES, LANES)]
        any_tail = jnp.max(idx_vec) >= TAIL_BASE

        @pl.when(any_tail)
        def _():
            pltpu.sync_copy(tokT_hbm.at[:, pl.ds(TAIL_BASE, VOCAB - TAIL_BASE)],
                            tail_v)
            for j in range(LANES):
                tok = idx_vec[j]

                @pl.when(tok >= TAIL_BASE)
                def _():
                    extract(tail_v, None, tok - TAIL_BASE, g * LANES + j)

        return 0

    lax.fori_loop(0, VGRP, tail_group, 0)

    pltpu.sync_copy(fbuf, out_hbm.at[wid])


TCB = 64
B_TC = BATCH * T_TC
TC_STEPS = B_TC // TCB
STEPS_PER_B = T_TC // TCB


STRIP_W = TCB * TILE_W + (VOCAB - TAIL_BASE)


def _tc_body(xs_ref, idx_ref, posT_ref, tokT_ref, tail_ref, out_ref, big, sem):
    s = pl.program_id(0)
    n = pl.num_programs(0)

    def fire(step, slot):
        for j in range(TCB):
            blk = pl.multiple_of(xs_ref[step * TCB + j], TILE_W)
            pltpu.make_async_copy(
                tokT_ref.at[:, pl.ds(blk, TILE_W)],
                big.at[slot, :, pl.ds(j * TILE_W, TILE_W)],
                sem.at[slot],
            ).start()

    def wait(slot):
        pltpu.make_async_copy(
            tokT_ref.at[:, pl.ds(0, TCB * TILE_W)],
            big.at[slot],
            sem.at[slot],
        ).wait()

    @pl.when(s == 0)
    def _():
        fire(0, 0)

    @pl.when(s + 1 < n)
    def _():
        fire(s + 1, (s + 1) % 2)

    wait(s % 2)

    ids = idx_ref[0, 0, :]
    safe = jnp.minimum(ids, SAFE_MAX)
    col = jnp.where(
        ids >= TAIL_BASE,
        TCB * TILE_W + ids - TAIL_BASE,
        (safe & (TILE_W - 1)) + lax.iota(jnp.int32, TCB) * TILE_W)
    onehot = (lax.broadcasted_iota(jnp.int32, (TCB, STRIP_W), 1)
              == col[:, None]).astype(jnp.float32)
    strip = jnp.concatenate([big[s % 2], tail_ref[...]], axis=1)
    cols = lax.dot_general(strip, onehot, (((1,), (1,)), ((), ())),
                           preferred_element_type=jnp.float32)
    out_ref[0, :, :] = cols + posT_ref[0]


@functools.partial(
    pl.pallas_call,
    grid_spec=pltpu.PrefetchScalarGridSpec(
        num_scalar_prefetch=1,
        grid=(TC_STEPS,),
        in_specs=[
            pl.BlockSpec((1, 1, TCB), lambda s, xs: (s, 0, 0)),
            pl.BlockSpec((1, N_EMBD, TCB), lambda s, xs: (s % STEPS_PER_B, 0, 0)),
            pl.BlockSpec(memory_space=pl.ANY),
            pl.BlockSpec((N_EMBD, VOCAB - TAIL_BASE), lambda s, xs: (0, 0)),
        ],
        out_specs=pl.BlockSpec((1, N_EMBD, TCB), lambda s, xs: (s, 0, 0)),
        scratch_shapes=[
            pltpu.VMEM((2, N_EMBD, TCB * TILE_W), jnp.float32),
            pltpu.SemaphoreType.DMA((2,)),
        ],
    ),
    out_shape=jax.ShapeDtypeStruct((TC_STEPS, N_EMBD, TCB), jnp.float32),
)
def _embed_tc(xs_ref, idx_ref, posT_ref, tokT_ref, tail_ref, out_ref, big, sem):
    _tc_body(xs_ref, idx_ref, posT_ref, tokT_ref, tail_ref, out_ref, big, sem)


def kernel(x, tok_table, pos_table):
    x = x.astype(jnp.int32)
    tokT = tok_table.T
    posT = pos_table.T
    x_sc = x[:, :T_SC].reshape(-1)
    x_tc = x[:, T_SC:].reshape(-1)
    pos3 = posT[:, T_SC:].reshape(N_EMBD, STEPS_PER_B, TCB).transpose(1, 0, 2)
    out_sc = _embed_sc(x_sc, tokT, posT)
    out_sc = (out_sc.reshape(BATCH, WPB, N_EMBD, BPW)
              .transpose(0, 2, 1, 3).reshape(BATCH, N_EMBD, T_SC))
    blk_tc = (jnp.minimum(x_tc, SAFE_MAX) >> 7) * TILE_W
    out_tc = _embed_tc(blk_tc, x_tc.reshape(TC_STEPS, 1, TCB), pos3, tokT,
                       tokT[:, TAIL_BASE:])
    out_tc = (out_tc.reshape(BATCH, STEPS_PER_B, N_EMBD, TCB)
              .transpose(0, 2, 1, 3).reshape(BATCH, N_EMBD, T_TC))
    out_fm = jnp.concatenate([out_sc, out_tc], axis=2)
    return out_fm.transpose(0, 2, 1)
